# Initial kernel scaffold; baseline (speedup 1.0000x reference)
import jax
import jax.numpy as jnp
from jax.experimental import pallas as pl


def kernel(location_x, loc_table, user_table, time_table, bandwidth):
    raise NotImplementedError("write your pallas kernel here")


# speedup vs baseline: 1.9100x; 1.9100x over previous
import functools

import jax
import jax.numpy as jnp
from jax import lax
from jax.experimental import pallas as pl
from jax.experimental.pallas import tpu as pltpu
from jax.experimental.pallas import tpu_sc as plsc

BATCH = 16384
HIST = 50
DIM = 64
B = BATCH * HIST
NUM_CORES = 2
NUM_SUBCORES = 16
NW = NUM_CORES * NUM_SUBCORES
B_PER_W = B // NW
CHUNK = 128
N_CHUNKS = B_PER_W // CHUNK
W = 8
N_OUTER = N_CHUNKS // W


def _gather_body(idx_hbm, table_hbm, out_hbm, idx_v, rows_v, gsem):
    wid = lax.axis_index("s") * NUM_CORES + lax.axis_index("c")
    base = wid * B_PER_W
    pltpu.sync_copy(idx_hbm.at[wid], idx_v)

    def outer(i, carry):
        copies = []
        for j in range(W):
            c = i * W + j
            copies.append(pltpu.async_copy(
                table_hbm.at[idx_v.at[c]],
                rows_v.at[pl.ds(j * CHUNK, CHUNK)],
                gsem))
        for cp in copies:
            cp.wait()
        pltpu.sync_copy(rows_v, out_hbm.at[pl.ds(base + i * (W * CHUNK), W * CHUNK)])
        return carry

    lax.fori_loop(0, N_OUTER, outer, 0)


@jax.jit
def _gather(idx, table):
    mesh = plsc.VectorSubcoreMesh(core_axis_name="c", subcore_axis_name="s")
    kfn = functools.partial(
        pl.kernel,
        mesh=mesh,
        out_type=jax.ShapeDtypeStruct((B, DIM), jnp.float32),
        scratch_types=[
            pltpu.VMEM((N_CHUNKS, CHUNK), jnp.int32),
            pltpu.VMEM((W * CHUNK, DIM), jnp.float32),
            pltpu.SemaphoreType.DMA,
        ],
        compiler_params=pltpu.CompilerParams(use_tc_tiling_on_sc=False),
    )(_gather_body)
    return kfn(idx, table)


def _smooth_body(time_ref, bw_ref, out_ref):
    t = lax.broadcasted_iota(jnp.int32, (24, 24), 1)
    tn = lax.broadcasted_iota(jnp.int32, (24, 24), 0)
    d = jnp.abs(t - tn).astype(jnp.float32)
    dist = jnp.minimum(d, 24.0 - d)
    z = dist / bw_ref[0]
    w = jnp.exp(-0.5 * z * z)
    out_ref[...] = jnp.dot(w, time_ref[...], preferred_element_type=jnp.float32)


@jax.jit
def _smooth(time_table, bandwidth):
    return pl.pallas_call(
        _smooth_body,
        out_shape=jax.ShapeDtypeStruct((24, DIM), jnp.float32),
        in_specs=[
            pl.BlockSpec(memory_space=pltpu.VMEM),
            pl.BlockSpec(memory_space=pltpu.SMEM),
        ],
        out_specs=pl.BlockSpec(memory_space=pltpu.VMEM),
    )(time_table, bandwidth)


def kernel(location_x, loc_table, user_table, time_table, bandwidth):
    idx = location_x.reshape(NW, N_CHUNKS, CHUNK)
    loc_flat = _gather(idx, loc_table)
    loc_embedded = loc_flat.reshape(BATCH, HIST, DIM)
    smoothed = _smooth(time_table, bandwidth)
    return (loc_embedded, time_table, smoothed, user_table)


# speedup vs baseline: 1.9262x; 1.0085x over previous
import functools

import jax
import jax.numpy as jnp
from jax import lax
from jax.experimental import pallas as pl
from jax.experimental.pallas import tpu as pltpu
from jax.experimental.pallas import tpu_sc as plsc

BATCH = 16384
HIST = 50
DIM = 64
B = BATCH * HIST
NUM_CORES = 2
NUM_SUBCORES = 16
NW = NUM_CORES * NUM_SUBCORES
B_PER_W = B // NW
CHUNK = 128
N_CHUNKS = B_PER_W // CHUNK
K = 8
D = 4


def _gather_body(idx_hbm, table_hbm, out_hbm, idx_v, *scr):
    bufs = scr[0:K]
    gsem = scr[K:2 * K]
    ssem = scr[2 * K:3 * K]

    wid = lax.axis_index("s") * NUM_CORES + lax.axis_index("c")
    base = wid * B_PER_W
    pltpu.sync_copy(idx_hbm.at[wid], idx_v)

    def fire_g(c, slot):
        pltpu.async_copy(table_hbm.at[idx_v.at[c]], bufs[slot], gsem[slot])

    def wait_g(c, slot):
        pltpu.make_async_copy(
            table_hbm.at[idx_v.at[c]], bufs[slot], gsem[slot]).wait()

    def fire_s(c, slot):
        pltpu.async_copy(
            bufs[slot], out_hbm.at[pl.ds(base + c * CHUNK, CHUNK)], ssem[slot])

    def wait_s(c, slot):
        pltpu.make_async_copy(
            bufs[slot], out_hbm.at[pl.ds(base + c * CHUNK, CHUNK)],
            ssem[slot]).wait()

    for c in range(K):
        fire_g(c, c)
        if c >= D:
            wait_g(c - D, c - D)
            fire_s(c - D, c - D)

    def outer(ii, carry):
        i = K + ii * K
        for b in range(K):
            c = i + b
            wait_s(c - K, b)
            fire_g(c, b)
            sb = (b + K - D) % K
            wait_g(c - D, sb)
            fire_s(c - D, sb)
        return carry

    lax.fori_loop(0, (N_CHUNKS - K) // K, outer, 0)

    for r in range(N_CHUNKS - D, N_CHUNKS):
        wait_g(r, r % K)
        fire_s(r, r % K)
    for r in range(N_CHUNKS - K, N_CHUNKS):
        wait_s(r, r % K)


@jax.jit
def _gather(idx, table):
    mesh = plsc.VectorSubcoreMesh(core_axis_name="c", subcore_axis_name="s")
    scratch = [pltpu.VMEM((N_CHUNKS, CHUNK), jnp.int32)]
    scratch += [pltpu.VMEM((CHUNK, DIM), jnp.float32) for _ in range(K)]
    scratch += [pltpu.SemaphoreType.DMA for _ in range(2 * K)]
    kfn = functools.partial(
        pl.kernel,
        mesh=mesh,
        out_type=jax.ShapeDtypeStruct((B, DIM), jnp.float32),
        scratch_types=scratch,
        compiler_params=pltpu.CompilerParams(use_tc_tiling_on_sc=False),
    )(_gather_body)
    return kfn(idx, table)


def _smooth_body(time_ref, bw_ref, out_ref):
    t = lax.broadcasted_iota(jnp.int32, (24, 24), 1)
    tn = lax.broadcasted_iota(jnp.int32, (24, 24), 0)
    d = jnp.abs(t - tn).astype(jnp.float32)
    dist = jnp.minimum(d, 24.0 - d)
    z = dist / bw_ref[0]
    w = jnp.exp(-0.5 * z * z)
    out_ref[...] = jnp.dot(w, time_ref[...], preferred_element_type=jnp.float32)


@jax.jit
def _smooth(time_table, bandwidth):
    return pl.pallas_call(
        _smooth_body,
        out_shape=jax.ShapeDtypeStruct((24, DIM), jnp.float32),
        in_specs=[
            pl.BlockSpec(memory_space=pltpu.VMEM),
            pl.BlockSpec(memory_space=pltpu.SMEM),
        ],
        out_specs=pl.BlockSpec(memory_space=pltpu.VMEM),
    )(time_table, bandwidth)


def kernel(location_x, loc_table, user_table, time_table, bandwidth):
    idx = location_x.reshape(NW, N_CHUNKS, CHUNK)
    loc_flat = _gather(idx, loc_table)
    loc_embedded = loc_flat.reshape(BATCH, HIST, DIM)
    smoothed = _smooth(time_table, bandwidth)
    return (loc_embedded, time_table, smoothed, user_table)
